# Initial kernel scaffold; baseline (speedup 1.0000x reference)
import jax
import jax.numpy as jnp
from jax.experimental import pallas as pl


def kernel(x, edge_index, edge_attr, WQ, WK, WV, WE1, WE2, bE2):
    raise NotImplementedError("write your pallas kernel here")


# speedup vs baseline: 6.1660x; 6.1660x over previous
import functools

import jax
import jax.numpy as jnp
from jax import lax
from jax.experimental import pallas as pl
from jax.experimental.pallas import tpu as pltpu
from jax.experimental.pallas import tpu_sc as plsc

N1 = 10000
N2 = 10000
DEG = 32
D = 128
E = N2 * DEG


def _proj_body(x_ref, w_ref, out_ref):
    out_ref[...] = jnp.dot(x_ref[...], w_ref[...],
                           preferred_element_type=jnp.float32)


def _proj(x, w):
    blk = 2000
    return pl.pallas_call(
        _proj_body,
        grid=(N1 // blk,),
        in_specs=[
            pl.BlockSpec((blk, D), lambda i: (i, 0)),
            pl.BlockSpec((D, w.shape[1]), lambda i: (0, 0)),
        ],
        out_specs=pl.BlockSpec((blk, w.shape[1]), lambda i: (i, 0)),
        out_shape=jax.ShapeDtypeStruct((N1, w.shape[1]), jnp.float32),
    )(x, w)


_SC_CHUNK = 200


def _sc_gather(table, idx):
    info = plsc.get_sparse_core_info()
    nc, ns = info.num_cores, info.num_subcores
    nw = nc * ns
    per_w = E // nw
    n_chunks = per_w // _SC_CHUNK
    width = table.shape[1]

    mesh = plsc.VectorSubcoreMesh(core_axis_name="c", subcore_axis_name="s")

    @functools.partial(
        pl.kernel,
        out_type=jax.ShapeDtypeStruct((E, width), table.dtype),
        mesh=mesh,
        scratch_types=[
            pltpu.VMEM((_SC_CHUNK,), jnp.int32),
            pltpu.VMEM((_SC_CHUNK, width), table.dtype),
            pltpu.SemaphoreType.DMA,
        ],
    )
    def k(table_hbm, idx_hbm, out_hbm, idx_v, rows_v, sem):
        wid = lax.axis_index("s") * nc + lax.axis_index("c")
        w_base = wid * per_w

        def body(i, carry):
            base = w_base + i * _SC_CHUNK
            pltpu.sync_copy(idx_hbm.at[pl.ds(base, _SC_CHUNK)], idx_v)
            pltpu.async_copy(table_hbm.at[idx_v], rows_v, sem).wait()
            pltpu.sync_copy(rows_v, out_hbm.at[pl.ds(base, _SC_CHUNK)])
            return carry

        lax.fori_loop(0, n_chunks, body, 0)

    return k(table, idx)


def _attn_body(x_ref, ea_ref, kvg_ref, wq_ref, we1_ref, we2y_ref, be2y_ref,
               out_ref):
    b = x_ref.shape[0]
    r = b * DEG
    f32 = jnp.float32

    row = lax.broadcasted_iota(jnp.int32, (r, D), 0) % 32
    lane = lax.broadcasted_iota(jnp.int32, (r, D), 1)
    s1 = lax.broadcasted_iota(jnp.int32, (D, D), 0)
    s2 = lax.broadcasted_iota(jnp.int32, (D, D), 1)

    t32 = (s1 % 32 == s2 % 32).astype(f32)
    gsum = (s1 // 32 == s2 // 32).astype(f32)
    gsum2 = ((s1 % 32) // 8 == (s2 % 32) // 8).astype(f32)
    m2 = (s2 // 32 == (s1 % 32) // 8).astype(f32)
    mask_h = (lane // 32 == row // 8).astype(f32)
    mask_j = (lane % 32 == row).astype(f32)
    mask_e2 = ((lane // 32 == row % 4) & (lane % 32 % 8 == row // 4)).astype(f32)
    mask_j2 = ((lane % 32 % 8) * 4 + lane // 32 == row).astype(f32)

    xb = x_ref[...]
    q = jnp.dot(xb, wq_ref[...], preferred_element_type=f32)
    q_rows = jnp.broadcast_to(q[:, None, :], (b, DEG, D)).reshape(r, D)
    qtile = jnp.dot(q_rows * mask_h, t32, preferred_element_type=f32)

    ea = ea_ref[...]
    e1 = jnp.dot(ea, we1_ref[...], preferred_element_type=f32)
    e2y = jnp.dot(ea, we2y_ref[...], preferred_element_type=f32) + be2y_ref[...]

    kg = kvg_ref[:, :D]
    vg = kvg_ref[:, D:]

    t = e1 * kg * qtile
    cs = jnp.dot(t, gsum, preferred_element_type=f32)
    contrib = cs * mask_j + e2y * mask_e2
    logits = jnp.clip(contrib.reshape(b, DEG, D).sum(axis=1), -8.0, 8.0)
    num = jnp.exp(logits)
    den = jnp.dot(num, gsum2, preferred_element_type=f32)
    p2 = num / den

    p_rows = jnp.broadcast_to(p2[:, None, :], (b, DEG, D)).reshape(r, D)
    w = jnp.dot(p_rows * mask_j2, m2, preferred_element_type=f32)
    out_ref[...] = (w * vg).reshape(b, DEG, D).sum(axis=1)


def _attn(x, edge_attr, kvg, WQ, WE1, WE2y, bE2y):
    b = 200
    r = b * DEG
    grid = N2 // b
    return pl.pallas_call(
        _attn_body,
        grid=(grid,),
        in_specs=[
            pl.BlockSpec((b, D), lambda i: (i, 0)),
            pl.BlockSpec((r, D), lambda i: (i, 0)),
            pl.BlockSpec((r, 2 * D), lambda i: (i, 0)),
            pl.BlockSpec((D, D), lambda i: (0, 0)),
            pl.BlockSpec((D, D), lambda i: (0, 0)),
            pl.BlockSpec((D, D), lambda i: (0, 0)),
            pl.BlockSpec((1, D), lambda i: (0, 0)),
        ],
        out_specs=pl.BlockSpec((b, D), lambda i: (i, 0)),
        out_shape=jax.ShapeDtypeStruct((N2, D), jnp.float32),
    )(x, edge_attr, kvg, WQ, WE1, WE2y, bE2y)


def kernel(x, edge_index, edge_attr, WQ, WK, WV, WE1, WE2, bE2):
    idx = edge_index[0].astype(jnp.int32)
    wkv = jnp.concatenate([WK, WV], axis=1)
    lanes = jnp.arange(D)
    we2y = WE2[:, (lanes % 32) // 8]
    be2y = bE2[(lanes % 32) // 8].reshape(1, D)
    kv = _proj(x, wkv)
    kvg = _sc_gather(kv, idx)
    return _attn(x, edge_attr, kvg, WQ, WE1, we2y, be2y)


# speedup vs baseline: 8.2462x; 1.3374x over previous
import functools

import jax
import jax.numpy as jnp
from jax import lax
from jax.experimental import pallas as pl
from jax.experimental.pallas import tpu as pltpu
from jax.experimental.pallas import tpu_sc as plsc

N1 = 10000
N2 = 10000
DEG = 32
D = 128
E = N2 * DEG


def _proj_body(x_ref, w_ref, out_ref):
    kv = jnp.dot(x_ref[...], w_ref[...], preferred_element_type=jnp.float32)
    kb = kv[:, :D].astype(jnp.bfloat16)
    vb = kv[:, D:].astype(jnp.bfloat16)
    ku = lax.bitcast_convert_type(kb, jnp.uint16).astype(jnp.uint32)
    vu = lax.bitcast_convert_type(vb, jnp.uint16).astype(jnp.uint32)
    out_ref[...] = lax.bitcast_convert_type((vu << 16) | ku, jnp.int32)


def _proj(x, w):
    blk = 2000
    return pl.pallas_call(
        _proj_body,
        grid=(N1 // blk,),
        in_specs=[
            pl.BlockSpec((blk, D), lambda i: (i, 0)),
            pl.BlockSpec((D, 2 * D), lambda i: (0, 0)),
        ],
        out_specs=pl.BlockSpec((blk, D), lambda i: (i, 0)),
        out_shape=jax.ShapeDtypeStruct((N1, D), jnp.int32),
    )(x, w)


_SC_CHUNK = 200


def _sc_gather(table, idx):
    info = plsc.get_sparse_core_info()
    nc, ns = info.num_cores, info.num_subcores
    nw = nc * ns
    per_w = E // nw
    n_pairs = per_w // (2 * _SC_CHUNK)
    width = table.shape[1]
    c = _SC_CHUNK

    mesh = plsc.VectorSubcoreMesh(core_axis_name="c", subcore_axis_name="s")

    @functools.partial(
        pl.kernel,
        out_type=jax.ShapeDtypeStruct((E, width), table.dtype),
        mesh=mesh,
        scratch_types=[
            pltpu.VMEM((c,), jnp.int32),
            pltpu.VMEM((c,), jnp.int32),
            pltpu.VMEM((c, width), table.dtype),
            pltpu.VMEM((c, width), table.dtype),
            pltpu.SemaphoreType.DMA,
            pltpu.SemaphoreType.DMA,
        ],
    )
    def k(table_hbm, idx_hbm, out_hbm, idx_a, idx_b, rows_a, rows_b,
          sem_a, sem_b):
        wid = lax.axis_index("s") * nc + lax.axis_index("c")
        w_base = wid * per_w

        def gather_a():
            return pltpu.make_async_copy(table_hbm.at[idx_a], rows_a, sem_a)

        def gather_b():
            return pltpu.make_async_copy(table_hbm.at[idx_b], rows_b, sem_b)

        pltpu.sync_copy(idx_hbm.at[pl.ds(w_base, c)], idx_a)
        gather_a().start()

        def body(i, carry):
            base_a = w_base + (2 * i) * c
            base_b = base_a + c
            pltpu.sync_copy(idx_hbm.at[pl.ds(base_b, c)], idx_b)
            gather_b().start()
            gather_a().wait()
            pltpu.sync_copy(rows_a, out_hbm.at[pl.ds(base_a, c)])

            @pl.when(i + 1 < n_pairs)
            def _():
                pltpu.sync_copy(idx_hbm.at[pl.ds(base_b + c, c)], idx_a)
                gather_a().start()

            gather_b().wait()
            pltpu.sync_copy(rows_b, out_hbm.at[pl.ds(base_b, c)])
            return carry

        lax.fori_loop(0, n_pairs, body, 0)

    return k(table, idx)


def _attn_body(x_ref, ea_ref, kvg_ref, wq_ref, we1_ref, we2y_ref, be2y_ref,
               out_ref):
    b = x_ref.shape[0]
    r = b * DEG
    f32 = jnp.float32
    bf16 = jnp.bfloat16

    row = lax.broadcasted_iota(jnp.int32, (r, D), 0) % 32
    lane = lax.broadcasted_iota(jnp.int32, (r, D), 1)
    s1 = lax.broadcasted_iota(jnp.int32, (D, D), 0)
    s2 = lax.broadcasted_iota(jnp.int32, (D, D), 1)

    t32 = (s1 % 32 == s2 % 32).astype(bf16)
    gsum = (s1 // 32 == s2 // 32).astype(bf16)
    gsum2 = ((s1 % 32) // 8 == (s2 % 32) // 8).astype(bf16)
    m2 = (s2 // 32 == (s1 % 32) // 8).astype(bf16)
    mask_h = (lane // 32 == row // 8).astype(bf16)
    mask_j = (lane % 32 == row).astype(f32)
    mask_e2 = ((lane // 32 == row % 4) & (lane % 32 % 8 == row // 4)).astype(f32)
    mask_j2 = ((lane % 32 % 8) * 4 + lane // 32 == row).astype(bf16)

    xb = x_ref[...].astype(bf16)
    q = jnp.dot(xb, wq_ref[...], preferred_element_type=f32)
    qb = q.astype(bf16)
    q_rows = jnp.broadcast_to(qb[:, None, :], (b, DEG, D)).reshape(r, D)
    qtile = jnp.dot(q_rows * mask_h, t32, preferred_element_type=f32)

    ea = ea_ref[...].astype(bf16)
    e1 = jnp.dot(ea, we1_ref[...], preferred_element_type=f32)
    e2y = jnp.dot(ea, we2y_ref[...], preferred_element_type=f32) + be2y_ref[...]

    kvg = lax.bitcast_convert_type(kvg_ref[...], jnp.uint32)
    kg = lax.bitcast_convert_type((kvg & 0xFFFF).astype(jnp.uint16), bf16)
    vg = lax.bitcast_convert_type((kvg >> 16).astype(jnp.uint16), bf16)

    t = (e1 * kg.astype(f32) * qtile).astype(bf16)
    cs = jnp.dot(t, gsum, preferred_element_type=f32)
    contrib = cs * mask_j + e2y * mask_e2
    logits = jnp.clip(contrib.reshape(b, DEG, D).sum(axis=1), -8.0, 8.0)
    num = jnp.exp(logits)
    den = jnp.dot(num.astype(bf16), gsum2, preferred_element_type=f32)
    p2 = num / den

    pb = p2.astype(bf16)
    p_rows = jnp.broadcast_to(pb[:, None, :], (b, DEG, D)).reshape(r, D)
    w = jnp.dot(p_rows * mask_j2, m2, preferred_element_type=f32)
    out_ref[...] = (w * vg.astype(f32)).reshape(b, DEG, D).sum(axis=1)


def _attn(x, edge_attr, kvg, WQ, WE1, WE2y, bE2y):
    b = 200
    r = b * DEG
    grid = N2 // b
    return pl.pallas_call(
        _attn_body,
        grid=(grid,),
        in_specs=[
            pl.BlockSpec((b, D), lambda i: (i, 0)),
            pl.BlockSpec((r, D), lambda i: (i, 0)),
            pl.BlockSpec((r, D), lambda i: (i, 0)),
            pl.BlockSpec((D, D), lambda i: (0, 0)),
            pl.BlockSpec((D, D), lambda i: (0, 0)),
            pl.BlockSpec((D, D), lambda i: (0, 0)),
            pl.BlockSpec((1, D), lambda i: (0, 0)),
        ],
        out_specs=pl.BlockSpec((b, D), lambda i: (i, 0)),
        out_shape=jax.ShapeDtypeStruct((N2, D), jnp.float32),
    )(x, edge_attr, kvg, WQ, WE1, WE2y, bE2y)


def kernel(x, edge_index, edge_attr, WQ, WK, WV, WE1, WE2, bE2):
    idx = edge_index[0].astype(jnp.int32)
    wkv = jnp.concatenate([WK, WV], axis=1)
    lanes = jnp.arange(D)
    we2y = WE2[:, (lanes % 32) // 8].astype(jnp.bfloat16)
    be2y = bE2[(lanes % 32) // 8].reshape(1, D)
    kv = _proj(x, wkv)
    kvg = _sc_gather(kv, idx)
    return _attn(x, edge_attr, kvg, WQ.astype(jnp.bfloat16),
                 WE1.astype(jnp.bfloat16), we2y, be2y)


# speedup vs baseline: 9.2980x; 1.1275x over previous
import functools

import jax
import jax.numpy as jnp
from jax import lax
from jax.experimental import pallas as pl
from jax.experimental.pallas import tpu as pltpu
from jax.experimental.pallas import tpu_sc as plsc

N1 = 10000
N2 = 10000
DEG = 32
D = 128
E = N2 * DEG


def _proj_body(x_ref, w_ref, out_ref):
    kv = jnp.dot(x_ref[...], w_ref[...], preferred_element_type=jnp.float32)
    kb = kv[:, :D].astype(jnp.bfloat16)
    vb = kv[:, D:].astype(jnp.bfloat16)
    ku = lax.bitcast_convert_type(kb, jnp.uint16).astype(jnp.uint32)
    vu = lax.bitcast_convert_type(vb, jnp.uint16).astype(jnp.uint32)
    out_ref[...] = lax.bitcast_convert_type((vu << 16) | ku, jnp.int32)


def _proj(x, w):
    blk = 2000
    return pl.pallas_call(
        _proj_body,
        grid=(N1 // blk,),
        in_specs=[
            pl.BlockSpec((blk, D), lambda i: (i, 0)),
            pl.BlockSpec((D, 2 * D), lambda i: (0, 0)),
        ],
        out_specs=pl.BlockSpec((blk, D), lambda i: (i, 0)),
        out_shape=jax.ShapeDtypeStruct((N1, D), jnp.int32),
    )(x, w)


_SC_CHUNK = 200


def _sc_gather(table, idx):
    info = plsc.get_sparse_core_info()
    nc, ns = info.num_cores, info.num_subcores
    nw = nc * ns
    per_w = E // nw
    n_pairs = per_w // (2 * _SC_CHUNK)
    width = table.shape[1]
    c = _SC_CHUNK

    mesh = plsc.VectorSubcoreMesh(core_axis_name="c", subcore_axis_name="s")

    @functools.partial(
        pl.kernel,
        out_type=jax.ShapeDtypeStruct((E, width), table.dtype),
        mesh=mesh,
        scratch_types=[
            pltpu.VMEM((c,), jnp.int32),
            pltpu.VMEM((c,), jnp.int32),
            pltpu.VMEM((c, width), table.dtype),
            pltpu.VMEM((c, width), table.dtype),
            pltpu.SemaphoreType.DMA,
            pltpu.SemaphoreType.DMA,
        ],
    )
    def k(table_hbm, idx_hbm, out_hbm, idx_a, idx_b, rows_a, rows_b,
          sem_a, sem_b):
        wid = lax.axis_index("s") * nc + lax.axis_index("c")
        w_base = wid * per_w

        def gather_a():
            return pltpu.make_async_copy(table_hbm.at[idx_a], rows_a, sem_a)

        def gather_b():
            return pltpu.make_async_copy(table_hbm.at[idx_b], rows_b, sem_b)

        pltpu.sync_copy(idx_hbm.at[pl.ds(w_base, c)], idx_a)
        gather_a().start()

        def body(i, carry):
            base_a = w_base + (2 * i) * c
            base_b = base_a + c
            pltpu.sync_copy(idx_hbm.at[pl.ds(base_b, c)], idx_b)
            gather_b().start()
            gather_a().wait()
            pltpu.sync_copy(rows_a, out_hbm.at[pl.ds(base_a, c)])

            @pl.when(i + 1 < n_pairs)
            def _():
                pltpu.sync_copy(idx_hbm.at[pl.ds(base_b + c, c)], idx_a)
                gather_a().start()

            gather_b().wait()
            pltpu.sync_copy(rows_b, out_hbm.at[pl.ds(base_b, c)])
            return carry

        lax.fori_loop(0, n_pairs, body, 0)

    return k(table, idx)


def _attn_body(x_ref, ea_ref, kvg_ref, wq_ref, we1_ref, we2y_ref, be2l_ref,
               t32_ref, gsum_ref, gsum2_ref, m2_ref,
               mh_ref, mj_ref, me2_ref, mj2_ref, out_ref):
    b = x_ref.shape[0]
    r = b * DEG
    f32 = jnp.float32
    bf16 = jnp.bfloat16

    xb = x_ref[...].astype(bf16)
    q = jnp.dot(xb, wq_ref[...], preferred_element_type=f32)
    qb = q.astype(bf16)
    q_rows = jnp.broadcast_to(qb[:, None, :], (b, DEG, D)).reshape(r, D)
    qtile = jnp.dot(q_rows * mh_ref[...], t32_ref[...],
                    preferred_element_type=f32)

    ea = ea_ref[...].astype(bf16)
    e1 = jnp.dot(ea, we1_ref[...], preferred_element_type=f32)
    e2y = jnp.dot(ea, we2y_ref[...], preferred_element_type=f32)

    kvg = lax.bitcast_convert_type(kvg_ref[...], jnp.uint32)
    kg = lax.bitcast_convert_type((kvg & 0xFFFF).astype(jnp.uint16), bf16)
    vg = lax.bitcast_convert_type((kvg >> 16).astype(jnp.uint16), bf16)

    t = (e1 * kg.astype(f32) * qtile).astype(bf16)
    cs = jnp.dot(t, gsum_ref[...], preferred_element_type=f32)
    contrib = cs * mj_ref[...] + e2y * me2_ref[...]
    logits = jnp.clip(contrib.reshape(b, DEG, D).sum(axis=1) + be2l_ref[...],
                      -8.0, 8.0)
    num = jnp.exp(logits)
    den = jnp.dot(num.astype(bf16), gsum2_ref[...],
                  preferred_element_type=f32)
    p2 = num / den

    pb = p2.astype(bf16)
    p_rows = jnp.broadcast_to(pb[:, None, :], (b, DEG, D)).reshape(r, D)
    w = jnp.dot(p_rows * mj2_ref[...], m2_ref[...],
                preferred_element_type=f32)
    out_ref[...] = (w * vg.astype(f32)).reshape(b, DEG, D).sum(axis=1)


_ATTN_B = 200


def _attn(x, edge_attr, kvg, WQ, WE1, WE2y, bE2l, consts):
    b = _ATTN_B
    r = b * DEG
    grid = N2 // b
    const_specs = [
        pl.BlockSpec(cc.shape, lambda i: tuple(0 for _ in cc.shape))
        for cc in consts
    ]
    return pl.pallas_call(
        _attn_body,
        grid=(grid,),
        in_specs=[
            pl.BlockSpec((b, D), lambda i: (i, 0)),
            pl.BlockSpec((r, D), lambda i: (i, 0)),
            pl.BlockSpec((r, D), lambda i: (i, 0)),
            pl.BlockSpec((D, D), lambda i: (0, 0)),
            pl.BlockSpec((D, D), lambda i: (0, 0)),
            pl.BlockSpec((D, D), lambda i: (0, 0)),
            pl.BlockSpec((1, D), lambda i: (0, 0)),
        ] + const_specs,
        out_specs=pl.BlockSpec((b, D), lambda i: (i, 0)),
        out_shape=jax.ShapeDtypeStruct((N2, D), jnp.float32),
    )(x, edge_attr, kvg, WQ, WE1, WE2y, bE2l, *consts)


def _make_consts():
    f32 = jnp.float32
    bf16 = jnp.bfloat16
    r = _ATTN_B * DEG
    row = jax.lax.broadcasted_iota(jnp.int32, (r, D), 0) % 32
    lane = jax.lax.broadcasted_iota(jnp.int32, (r, D), 1)
    s1 = jax.lax.broadcasted_iota(jnp.int32, (D, D), 0)
    s2 = jax.lax.broadcasted_iota(jnp.int32, (D, D), 1)

    t32 = (s1 % 32 == s2 % 32).astype(bf16)
    gsum = (s1 // 32 == s2 // 32).astype(bf16)
    gsum2 = ((s1 % 32) // 8 == (s2 % 32) // 8).astype(bf16)
    m2 = (s2 // 32 == (s1 % 32) // 8).astype(bf16)
    mask_h = (lane // 32 == row // 8).astype(bf16)
    mask_j = (lane % 32 == row).astype(f32)
    mask_e2 = ((lane // 32 == row % 4)
               & (lane % 32 % 8 == row // 4)).astype(f32)
    mask_j2 = ((lane % 32 % 8) * 4 + lane // 32 == row).astype(bf16)
    return (t32, gsum, gsum2, m2, mask_h, mask_j, mask_e2, mask_j2)


def kernel(x, edge_index, edge_attr, WQ, WK, WV, WE1, WE2, bE2):
    idx = edge_index[0].astype(jnp.int32)
    wkv = jnp.concatenate([WK, WV], axis=1)
    lanes = jnp.arange(D)
    we2y = WE2[:, (lanes % 32) // 8].astype(jnp.bfloat16)
    be2l = bE2[(lanes % 32) // 8].reshape(1, D)
    kv = _proj(x, wkv)
    kvg = _sc_gather(kv, idx)
    return _attn(x, edge_attr, kvg, WQ.astype(jnp.bfloat16),
                 WE1.astype(jnp.bfloat16), we2y, be2l, _make_consts())


# speedup vs baseline: 9.4586x; 1.0173x over previous
import functools

import jax
import jax.numpy as jnp
from jax import lax
from jax.experimental import pallas as pl
from jax.experimental.pallas import tpu as pltpu
from jax.experimental.pallas import tpu_sc as plsc

N1 = 10000
N2 = 10000
DEG = 32
D = 128
E = N2 * DEG

N_SLICES = 5
NODES_PER_SLICE = N2 // N_SLICES
EDGES_PER_SLICE = NODES_PER_SLICE * DEG


def _proj_body(x_ref, w_ref, out_ref):
    kv = jnp.dot(x_ref[...], w_ref[...], preferred_element_type=jnp.float32)
    kb = kv[:, :D].astype(jnp.bfloat16)
    vb = kv[:, D:].astype(jnp.bfloat16)
    ku = lax.bitcast_convert_type(kb, jnp.uint16).astype(jnp.uint32)
    vu = lax.bitcast_convert_type(vb, jnp.uint16).astype(jnp.uint32)
    out_ref[...] = lax.bitcast_convert_type((vu << 16) | ku, jnp.int32)


def _proj(x, w):
    blk = 2000
    return pl.pallas_call(
        _proj_body,
        grid=(N1 // blk,),
        in_specs=[
            pl.BlockSpec((blk, D), lambda i: (i, 0)),
            pl.BlockSpec((D, 2 * D), lambda i: (0, 0)),
        ],
        out_specs=pl.BlockSpec((blk, D), lambda i: (i, 0)),
        out_shape=jax.ShapeDtypeStruct((N1, D), jnp.int32),
    )(x, w)


_SC_CHUNK = 200


def _sc_gather(table, idx, n_edges):
    info = plsc.get_sparse_core_info()
    nc, ns = info.num_cores, info.num_subcores
    nw = nc * ns
    per_w = n_edges // nw
    n_pairs = per_w // (2 * _SC_CHUNK)
    width = table.shape[1]
    c = _SC_CHUNK

    mesh = plsc.VectorSubcoreMesh(core_axis_name="c", subcore_axis_name="s")

    @functools.partial(
        pl.kernel,
        out_type=jax.ShapeDtypeStruct((n_edges, width), table.dtype),
        mesh=mesh,
        scratch_types=[
            pltpu.VMEM((c,), jnp.int32),
            pltpu.VMEM((c,), jnp.int32),
            pltpu.VMEM((c, width), table.dtype),
            pltpu.VMEM((c, width), table.dtype),
            pltpu.SemaphoreType.DMA,
            pltpu.SemaphoreType.DMA,
        ],
    )
    def k(table_hbm, idx_hbm, out_hbm, idx_a, idx_b, rows_a, rows_b,
          sem_a, sem_b):
        wid = lax.axis_index("s") * nc + lax.axis_index("c")
        w_base = wid * per_w

        def gather_a():
            return pltpu.make_async_copy(table_hbm.at[idx_a], rows_a, sem_a)

        def gather_b():
            return pltpu.make_async_copy(table_hbm.at[idx_b], rows_b, sem_b)

        pltpu.sync_copy(idx_hbm.at[pl.ds(w_base, c)], idx_a)
        gather_a().start()

        def body(i, carry):
            base_a = w_base + (2 * i) * c
            base_b = base_a + c
            pltpu.sync_copy(idx_hbm.at[pl.ds(base_b, c)], idx_b)
            gather_b().start()
            gather_a().wait()
            pltpu.sync_copy(rows_a, out_hbm.at[pl.ds(base_a, c)])

            @pl.when(i + 1 < n_pairs)
            def _():
                pltpu.sync_copy(idx_hbm.at[pl.ds(base_b + c, c)], idx_a)
                gather_a().start()

            gather_b().wait()
            pltpu.sync_copy(rows_b, out_hbm.at[pl.ds(base_b, c)])
            return carry

        lax.fori_loop(0, n_pairs, body, 0)

    return k(table, idx)


def _attn_body(x_ref, ea_ref, kvg_ref, wq_ref, we1_ref, we2y_ref, be2l_ref,
               t32_ref, gsum_ref, gsum2_ref, m2_ref,
               mh_ref, mj_ref, me2_ref, mj2_ref, out_ref):
    b = x_ref.shape[0]
    r = b * DEG
    f32 = jnp.float32
    bf16 = jnp.bfloat16

    xb = x_ref[...].astype(bf16)
    q = jnp.dot(xb, wq_ref[...], preferred_element_type=f32)
    qb = q.astype(bf16)
    q_rows = jnp.broadcast_to(qb[:, None, :], (b, DEG, D)).reshape(r, D)
    qtile = jnp.dot(q_rows * mh_ref[...], t32_ref[...],
                    preferred_element_type=f32)

    ea = ea_ref[...].astype(bf16)
    e1 = jnp.dot(ea, we1_ref[...], preferred_element_type=f32)
    e2y = jnp.dot(ea, we2y_ref[...], preferred_element_type=f32)

    kvg = lax.bitcast_convert_type(kvg_ref[...], jnp.uint32)
    kg = lax.bitcast_convert_type((kvg & 0xFFFF).astype(jnp.uint16), bf16)
    vg = lax.bitcast_convert_type((kvg >> 16).astype(jnp.uint16), bf16)

    t = (e1 * qtile).astype(bf16) * kg
    cs = jnp.dot(t, gsum_ref[...], preferred_element_type=f32)
    contrib = cs * mj_ref[...] + e2y * me2_ref[...]
    logits = jnp.clip(contrib.reshape(b, DEG, D).sum(axis=1) + be2l_ref[...],
                      -8.0, 8.0)
    num = jnp.exp(logits)
    den = jnp.dot(num.astype(bf16), gsum2_ref[...],
                  preferred_element_type=f32)
    p2 = num / den

    pb = p2.astype(bf16)
    p_rows = jnp.broadcast_to(pb[:, None, :], (b, DEG, D)).reshape(r, D)
    w = jnp.dot(p_rows * mj2_ref[...], m2_ref[...],
                preferred_element_type=f32)
    out_ref[...] = (w * vg.astype(f32)).reshape(b, DEG, D).sum(axis=1)


_ATTN_B = 200


def _attn_slice(x, edge_attr, kvg_s, weights, consts, s):
    b = _ATTN_B
    r = b * DEG
    grid = NODES_PER_SLICE // b
    node_off = s * (NODES_PER_SLICE // b)
    edge_off = node_off
    const_specs = [
        pl.BlockSpec(cc.shape, lambda i: tuple(0 for _ in cc.shape))
        for cc in consts
    ]
    wq, we1, we2y, be2l = weights
    return pl.pallas_call(
        _attn_body,
        grid=(grid,),
        in_specs=[
            pl.BlockSpec((b, D), lambda i: (node_off + i, 0)),
            pl.BlockSpec((r, D), lambda i: (edge_off + i, 0)),
            pl.BlockSpec((r, D), lambda i: (i, 0)),
            pl.BlockSpec((D, D), lambda i: (0, 0)),
            pl.BlockSpec((D, D), lambda i: (0, 0)),
            pl.BlockSpec((D, D), lambda i: (0, 0)),
            pl.BlockSpec((1, D), lambda i: (0, 0)),
        ] + const_specs,
        out_specs=pl.BlockSpec((b, D), lambda i: (i, 0)),
        out_shape=jax.ShapeDtypeStruct((NODES_PER_SLICE, D), jnp.float32),
    )(x, edge_attr, kvg_s, wq, we1, we2y, be2l, *consts)


def _make_consts():
    f32 = jnp.float32
    bf16 = jnp.bfloat16
    r = _ATTN_B * DEG
    row = jax.lax.broadcasted_iota(jnp.int32, (r, D), 0) % 32
    lane = jax.lax.broadcasted_iota(jnp.int32, (r, D), 1)
    s1 = jax.lax.broadcasted_iota(jnp.int32, (D, D), 0)
    s2 = jax.lax.broadcasted_iota(jnp.int32, (D, D), 1)

    t32 = (s1 % 32 == s2 % 32).astype(bf16)
    gsum = (s1 // 32 == s2 // 32).astype(bf16)
    gsum2 = ((s1 % 32) // 8 == (s2 % 32) // 8).astype(bf16)
    m2 = (s2 // 32 == (s1 % 32) // 8).astype(bf16)
    mask_h = (lane // 32 == row // 8).astype(bf16)
    mask_j = (lane % 32 == row).astype(f32)
    mask_e2 = ((lane // 32 == row % 4)
               & (lane % 32 % 8 == row // 4)).astype(f32)
    mask_j2 = ((lane % 32 % 8) * 4 + lane // 32 == row).astype(bf16)
    return (t32, gsum, gsum2, m2, mask_h, mask_j, mask_e2, mask_j2)


def kernel(x, edge_index, edge_attr, WQ, WK, WV, WE1, WE2, bE2):
    idx = edge_index[0].astype(jnp.int32)
    wkv = jnp.concatenate([WK, WV], axis=1)
    lanes = jnp.arange(D)
    we2y = WE2[:, (lanes % 32) // 8].astype(jnp.bfloat16)
    be2l = bE2[(lanes % 32) // 8].reshape(1, D)
    weights = (WQ.astype(jnp.bfloat16), WE1.astype(jnp.bfloat16), we2y, be2l)
    consts = _make_consts()

    kv = _proj(x, wkv)
    outs = []
    for s in range(N_SLICES):
        idx_s = lax.dynamic_slice_in_dim(idx, s * EDGES_PER_SLICE,
                                         EDGES_PER_SLICE)
        kvg_s = _sc_gather(kv, idx_s, EDGES_PER_SLICE)
        outs.append(_attn_slice(x, edge_attr, kvg_s, weights, consts, s))
    return jnp.concatenate(outs, axis=0)
